# baseline (speedup 1.0000x reference)
import functools

import jax
import jax.numpy as jnp
from jax import lax
from jax.experimental import pallas as pl
from jax.experimental.pallas import tpu as pltpu
from jax.experimental.pallas import tpu_sc as plsc

EMB = 128
E = 320000
NW = 32
SUPER = 256
GROUPS = SUPER // 16
NSUPER = E // SUPER
T_FULL = NSUPER // NW
NLEFT = NSUPER - T_FULL * NW


def _table_body(w0_ref, w1_ref, w2_ref, t_ref):
    for i in range(4):
        for j in range(2):
            base = i * 12 + j * 6
            t_ref[base:base + 6, :] = (
                w2_ref[:, :] + w0_ref[i:i + 1, :] + w1_ref[j:j + 1, :]
            )


def _build_table(w0, w1, w2):
    return pl.pallas_call(
        _table_body,
        out_shape=jax.ShapeDtypeStruct((48, EMB), jnp.float32),
    )(w0, w1, w2)


def _sc_body(a0_hbm, a1_hbm, a2_hbm, table_hbm, out_hbm,
             table_v, a0_0, a1_0, a2_0, a0_1, a1_1, a2_1, a0_2, a1_2, a2_2,
             rows0_v, rows1_v, rows2_v,
             a_sem0, a_sem1, a_sem2, o_sem0, o_sem1, o_sem2, t_sem):
    wid = lax.axis_index("s") * 2 + lax.axis_index("c")
    a_sems = (a_sem0, a_sem1, a_sem2)
    o_sems = (o_sem0, o_sem1, o_sem2)
    rows = (rows0_v, rows1_v, rows2_v)
    attrs = ((a0_0, a1_0, a2_0), (a0_1, a1_1, a2_1), (a0_2, a1_2, a2_2))

    def cid_of(t):
        return t * NW + wid

    def attr_copies(t, b):
        base = cid_of(t) * SUPER
        return [
            pltpu.make_async_copy(a0_hbm.at[pl.ds(base, SUPER)], attrs[b][0], a_sems[b]),
            pltpu.make_async_copy(a1_hbm.at[pl.ds(base, SUPER)], attrs[b][1], a_sems[b]),
            pltpu.make_async_copy(a2_hbm.at[pl.ds(base, SUPER)], attrs[b][2], a_sems[b]),
        ]

    def issue_attr(t, b):
        for c in attr_copies(t, b):
            c.start()

    def wait_attr(t, b):
        for c in attr_copies(t, b):
            c.wait()

    def out_copy(t, b):
        base = cid_of(t) * SUPER * EMB
        return pltpu.make_async_copy(
            rows[b], out_hbm.at[pl.ds(base, SUPER * EMB)], o_sems[b])

    def expand(b):
        rb = rows[b]

        def group(g, carry):
            sl = pl.ds(g * 16, 16)
            code = (attrs[b][0][sl] * 12 + attrs[b][1][sl] * 6 + attrs[b][2][sl]) * EMB
            dst0 = g * (16 * EMB)

            def loads(r):
                src = code[r]
                return [table_v[pl.ds(src + s * 16, 16)] for s in range(8)]

            def stores(r, vals):
                dstb = dst0 + r * EMB
                for s in range(8):
                    rb[pl.ds(dstb + s * 16, 16)] = vals[s]

            def loads_zip_stores(r_load, r_store, vals_cur):
                src = code[r_load]
                dstb = dst0 + r_store * EMB
                vals_next = []
                for s in range(8):
                    vals_next.append(table_v[pl.ds(src + s * 16, 16)])
                    rb[pl.ds(dstb + s * 16, 16)] = vals_cur[s]
                return vals_next

            vals_cur = loads(0)
            for r in range(1, 16):
                vals_cur = loads_zip_stores(r, r - 1, vals_cur)
            stores(15, vals_cur)
            return carry

        lax.fori_loop(0, GROUPS, group, 0, unroll=2)

    pltpu.async_copy(table_hbm, table_v, t_sem).wait()

    def section(t, b, *, first=False, static_tail=False):
        wait_attr(t, b)
        if not first:
            out_copy(t - 3, b).wait()
        expand(b)
        if not static_tail:
            issue_attr(t + 3, b)
        out_copy(t, b).start()

    issue_attr(0, 0)
    issue_attr(1, 1)
    issue_attr(2, 2)
    section(0, 0, first=True)
    section(1, 1, first=True)
    section(2, 2, first=True)

    def loop_body(j, carry):
        t0 = j * 3
        section(t0, 0)
        section(t0 + 1, 1)
        section(t0 + 2, 2)
        return carry

    lax.fori_loop(1, 12, loop_body, 0)

    section(36, 0, static_tail=True)
    section(37, 1, static_tail=True)
    section(38, 2, static_tail=True)

    out_copy(36, 0).wait()
    out_copy(37, 1).wait()
    out_copy(38, 2).wait()

    @pl.when(wid < NLEFT)
    def _():
        t_extra = T_FULL
        issue_attr(t_extra, 0)
        wait_attr(t_extra, 0)
        expand(0)
        out_copy(t_extra, 0).start()
        out_copy(t_extra, 0).wait()


_sc_gather = functools.partial(
    pl.kernel,
    out_type=jax.ShapeDtypeStruct((E * EMB,), jnp.float32),
    mesh=plsc.VectorSubcoreMesh(core_axis_name="c", subcore_axis_name="s"),
    compiler_params=pltpu.CompilerParams(needs_layout_passes=False),
    scratch_types=[
        pltpu.VMEM((48 * EMB,), jnp.float32),
        pltpu.VMEM((SUPER,), jnp.int32),
        pltpu.VMEM((SUPER,), jnp.int32),
        pltpu.VMEM((SUPER,), jnp.int32),
        pltpu.VMEM((SUPER,), jnp.int32),
        pltpu.VMEM((SUPER,), jnp.int32),
        pltpu.VMEM((SUPER,), jnp.int32),
        pltpu.VMEM((SUPER,), jnp.int32),
        pltpu.VMEM((SUPER,), jnp.int32),
        pltpu.VMEM((SUPER,), jnp.int32),
        pltpu.VMEM((SUPER * EMB,), jnp.float32),
        pltpu.VMEM((SUPER * EMB,), jnp.float32),
        pltpu.VMEM((SUPER * EMB,), jnp.float32),
        pltpu.SemaphoreType.DMA,
        pltpu.SemaphoreType.DMA,
        pltpu.SemaphoreType.DMA,
        pltpu.SemaphoreType.DMA,
        pltpu.SemaphoreType.DMA,
        pltpu.SemaphoreType.DMA,
        pltpu.SemaphoreType.DMA,
    ],
)(_sc_body)


@jax.jit
def kernel(edge_attr, W0, W1, W2):
    a = edge_attr.astype(jnp.int32)
    table = _build_table(W0, W1, W2).reshape(-1)
    flat = _sc_gather(a[:, 0], a[:, 1], a[:, 2], table)
    return flat.reshape(E, EMB)


# speedup vs baseline: 1.3746x; 1.3746x over previous
import functools

import jax
import jax.numpy as jnp
from jax import lax
from jax.experimental import pallas as pl
from jax.experimental.pallas import tpu as pltpu
from jax.experimental.pallas import tpu_sc as plsc

EMB = 128
E = 320000
NW = 32
SUPER = 256
GROUPS = SUPER // 16
NSUPER = E // SUPER
T_FULL = NSUPER // NW
NLEFT = NSUPER - T_FULL * NW


def _table_body(w0_ref, w1_ref, w2_ref, t_ref):
    for i in range(4):
        for j in range(2):
            base = i * 12 + j * 6
            t_ref[base:base + 6, :] = (
                w2_ref[:, :] + w0_ref[i:i + 1, :] + w1_ref[j:j + 1, :]
            )


def _build_table(w0, w1, w2):
    return pl.pallas_call(
        _table_body,
        out_shape=jax.ShapeDtypeStruct((48, EMB), jnp.float32),
    )(w0, w1, w2)


def _sc_body(a0_hbm, a1_hbm, a2_hbm, table_hbm, out_hbm,
             table_v, a0_0, a1_0, a2_0, a0_1, a1_1, a2_1, a0_2, a1_2, a2_2,
             rows0_v, rows1_v, rows2_v,
             a_sem0, a_sem1, a_sem2, o_sem0, o_sem1, o_sem2, t_sem):
    wid = lax.axis_index("s") * 2 + lax.axis_index("c")
    a_sems = (a_sem0, a_sem1, a_sem2)
    o_sems = (o_sem0, o_sem1, o_sem2)
    rows = (rows0_v, rows1_v, rows2_v)
    attrs = ((a0_0, a1_0, a2_0), (a0_1, a1_1, a2_1), (a0_2, a1_2, a2_2))

    def cid_of(t):
        return t * NW + wid

    def attr_copies(t, b):
        base = cid_of(t) * SUPER
        return [
            pltpu.make_async_copy(a0_hbm.at[pl.ds(base, SUPER)],
                                  attrs[b][0].at[pl.ds(0, SUPER)], a_sems[b]),
            pltpu.make_async_copy(a1_hbm.at[pl.ds(base, SUPER)],
                                  attrs[b][1].at[pl.ds(0, SUPER)], a_sems[b]),
            pltpu.make_async_copy(a2_hbm.at[pl.ds(base, SUPER)],
                                  attrs[b][2].at[pl.ds(0, SUPER)], a_sems[b]),
        ]

    def issue_attr(t, b):
        for c in attr_copies(t, b):
            c.start()

    def wait_attr(t, b):
        for c in attr_copies(t, b):
            c.wait()

    def out_copy(t, b):
        base = cid_of(t) * SUPER * EMB
        return pltpu.make_async_copy(
            rows[b], out_hbm.at[pl.ds(base, SUPER * EMB)], o_sems[b])

    def expand(b):
        rb = rows[b]
        a0, a1, a2 = attrs[b]

        def codes_of(g):
            sl = pl.ds(g * 16, 16)
            cv = (a0[sl] * 12 + a1[sl] * 6 + a2[sl]) * EMB
            return tuple(cv[r] for r in range(16))

        def group(g, codes):
            dst0 = g * (16 * EMB)
            next_codes = codes_of(g + 1)

            def loads(r):
                src = codes[r]
                return [table_v[pl.ds(src + s * 16, 16)] for s in range(8)]

            def stores(r, vals):
                dstb = dst0 + r * EMB
                for s in range(8):
                    rb[pl.ds(dstb + s * 16, 16)] = vals[s]

            def loads_zip_stores(r_load, r_store, vals_cur):
                src = codes[r_load]
                dstb = dst0 + r_store * EMB
                vals_next = []
                for s in range(8):
                    vals_next.append(table_v[pl.ds(src + s * 16, 16)])
                    rb[pl.ds(dstb + s * 16, 16)] = vals_cur[s]
                return vals_next

            vals_cur = loads(0)
            for r in range(1, 16):
                vals_cur = loads_zip_stores(r, r - 1, vals_cur)
            stores(15, vals_cur)
            return next_codes

        lax.fori_loop(0, GROUPS, group, codes_of(0))

    pltpu.async_copy(table_hbm, table_v, t_sem).wait()

    def section(t, b, *, first=False, static_tail=False):
        wait_attr(t, b)
        if not first:
            out_copy(t - 3, b).wait()
        expand(b)
        if not static_tail:
            issue_attr(t + 3, b)
        out_copy(t, b).start()

    issue_attr(0, 0)
    issue_attr(1, 1)
    issue_attr(2, 2)
    section(0, 0, first=True)
    section(1, 1, first=True)
    section(2, 2, first=True)

    def loop_body(j, carry):
        t0 = j * 3
        section(t0, 0)
        section(t0 + 1, 1)
        section(t0 + 2, 2)
        return carry

    lax.fori_loop(1, 12, loop_body, 0)

    section(36, 0, static_tail=True)
    section(37, 1, static_tail=True)
    section(38, 2, static_tail=True)

    out_copy(36, 0).wait()
    out_copy(37, 1).wait()
    out_copy(38, 2).wait()

    @pl.when(wid < NLEFT)
    def _():
        t_extra = T_FULL
        issue_attr(t_extra, 0)
        wait_attr(t_extra, 0)
        expand(0)
        out_copy(t_extra, 0).start()
        out_copy(t_extra, 0).wait()


_sc_gather = functools.partial(
    pl.kernel,
    out_type=jax.ShapeDtypeStruct((E * EMB,), jnp.float32),
    mesh=plsc.VectorSubcoreMesh(core_axis_name="c", subcore_axis_name="s"),
    compiler_params=pltpu.CompilerParams(needs_layout_passes=False),
    scratch_types=[
        pltpu.VMEM((48 * EMB,), jnp.float32),
        pltpu.VMEM((SUPER + 16,), jnp.int32),
        pltpu.VMEM((SUPER + 16,), jnp.int32),
        pltpu.VMEM((SUPER + 16,), jnp.int32),
        pltpu.VMEM((SUPER + 16,), jnp.int32),
        pltpu.VMEM((SUPER + 16,), jnp.int32),
        pltpu.VMEM((SUPER + 16,), jnp.int32),
        pltpu.VMEM((SUPER + 16,), jnp.int32),
        pltpu.VMEM((SUPER + 16,), jnp.int32),
        pltpu.VMEM((SUPER + 16,), jnp.int32),
        pltpu.VMEM((SUPER * EMB,), jnp.float32),
        pltpu.VMEM((SUPER * EMB,), jnp.float32),
        pltpu.VMEM((SUPER * EMB,), jnp.float32),
        pltpu.SemaphoreType.DMA,
        pltpu.SemaphoreType.DMA,
        pltpu.SemaphoreType.DMA,
        pltpu.SemaphoreType.DMA,
        pltpu.SemaphoreType.DMA,
        pltpu.SemaphoreType.DMA,
        pltpu.SemaphoreType.DMA,
    ],
)(_sc_body)


@jax.jit
def kernel(edge_attr, W0, W1, W2):
    a = edge_attr.astype(jnp.int32)
    table = _build_table(W0, W1, W2).reshape(-1)
    flat = _sc_gather(a[:, 0], a[:, 1], a[:, 2], table)
    return flat.reshape(E, EMB)


# speedup vs baseline: 1.3943x; 1.0144x over previous
import functools

import jax
import jax.numpy as jnp
from jax import lax
from jax.experimental import pallas as pl
from jax.experimental.pallas import tpu as pltpu
from jax.experimental.pallas import tpu_sc as plsc

EMB = 128
E = 320000
NW = 32
SUPER = 256
GROUPS = SUPER // 16
NSUPER = E // SUPER
T_FULL = NSUPER // NW
NLEFT = NSUPER - T_FULL * NW


def _table_body(w0_ref, w1_ref, w2_ref, t_ref):
    for i in range(4):
        for j in range(2):
            base = i * 12 + j * 6
            t_ref[base:base + 6, :] = (
                w2_ref[:, :] + w0_ref[i:i + 1, :] + w1_ref[j:j + 1, :]
            )


def _build_table(w0, w1, w2):
    return pl.pallas_call(
        _table_body,
        out_shape=jax.ShapeDtypeStruct((48, EMB), jnp.float32),
    )(w0, w1, w2)


def _sc_body(a0_hbm, a1_hbm, a2_hbm, table_hbm, out_hbm,
             table_v, a0_0, a1_0, a2_0, a0_1, a1_1, a2_1, a0_2, a1_2, a2_2,
             rows0_v, rows1_v, rows2_v,
             a_sem0, a_sem1, a_sem2, o_sem0, o_sem1, o_sem2, t_sem):
    wid = lax.axis_index("s") * 2 + lax.axis_index("c")
    a_sems = (a_sem0, a_sem1, a_sem2)
    o_sems = (o_sem0, o_sem1, o_sem2)
    rows = (rows0_v, rows1_v, rows2_v)
    attrs = ((a0_0, a1_0, a2_0), (a0_1, a1_1, a2_1), (a0_2, a1_2, a2_2))

    def cid_of(t):
        return t * NW + wid

    def attr_copies(t, b):
        base = cid_of(t) * SUPER
        return [
            pltpu.make_async_copy(a0_hbm.at[pl.ds(base, SUPER)],
                                  attrs[b][0].at[pl.ds(0, SUPER)], a_sems[b]),
            pltpu.make_async_copy(a1_hbm.at[pl.ds(base, SUPER)],
                                  attrs[b][1].at[pl.ds(0, SUPER)], a_sems[b]),
            pltpu.make_async_copy(a2_hbm.at[pl.ds(base, SUPER)],
                                  attrs[b][2].at[pl.ds(0, SUPER)], a_sems[b]),
        ]

    def issue_attr(t, b):
        for c in attr_copies(t, b):
            c.start()

    def wait_attr(t, b):
        for c in attr_copies(t, b):
            c.wait()

    def out_copy(t, b):
        base = cid_of(t) * SUPER * EMB
        return pltpu.make_async_copy(
            rows[b], out_hbm.at[pl.ds(base, SUPER * EMB)], o_sems[b])

    def expand(b):
        rb = rows[b]
        a0, a1, a2 = attrs[b]

        def codes_of(g):
            sl = pl.ds(g * 16, 16)
            cv = (a0[sl] * 12 + a1[sl] * 6 + a2[sl]) * EMB
            return tuple(cv[r] for r in range(16))

        def group(g, codes):
            dst0 = g * (16 * EMB)
            next_codes = codes_of(g + 1)

            def loads(r):
                src = codes[r]
                return [table_v[pl.ds(src + s * 16, 16)] for s in range(8)]

            def stores(r, vals):
                dstb = dst0 + r * EMB
                for s in range(8):
                    rb[pl.ds(dstb + s * 16, 16)] = vals[s]

            def loads_zip_stores(r_load, r_store, vals_cur):
                src = codes[r_load]
                dstb = dst0 + r_store * EMB
                vals_next = []
                for s in range(8):
                    vals_next.append(table_v[pl.ds(src + s * 16, 16)])
                    rb[pl.ds(dstb + s * 16, 16)] = vals_cur[s]
                return vals_next

            vals_cur = loads(0)
            for r in range(1, 16):
                vals_cur = loads_zip_stores(r, r - 1, vals_cur)
            stores(15, vals_cur)
            return next_codes

        lax.fori_loop(0, GROUPS, group, codes_of(0), unroll=2)

    pltpu.async_copy(table_hbm, table_v, t_sem).wait()

    def section(t, b, *, first=False, static_tail=False):
        wait_attr(t, b)
        if not first:
            out_copy(t - 3, b).wait()
        expand(b)
        if not static_tail:
            issue_attr(t + 3, b)
        out_copy(t, b).start()

    issue_attr(0, 0)
    issue_attr(1, 1)
    issue_attr(2, 2)
    section(0, 0, first=True)
    section(1, 1, first=True)
    section(2, 2, first=True)

    def loop_body(j, carry):
        t0 = j * 3
        section(t0, 0)
        section(t0 + 1, 1)
        section(t0 + 2, 2)
        return carry

    lax.fori_loop(1, 12, loop_body, 0)

    section(36, 0, static_tail=True)
    section(37, 1, static_tail=True)
    section(38, 2, static_tail=True)

    out_copy(36, 0).wait()
    out_copy(37, 1).wait()
    out_copy(38, 2).wait()

    @pl.when(wid < NLEFT)
    def _():
        t_extra = T_FULL
        issue_attr(t_extra, 0)
        wait_attr(t_extra, 0)
        expand(0)
        out_copy(t_extra, 0).start()
        out_copy(t_extra, 0).wait()


_sc_gather = functools.partial(
    pl.kernel,
    out_type=jax.ShapeDtypeStruct((E * EMB,), jnp.float32),
    mesh=plsc.VectorSubcoreMesh(core_axis_name="c", subcore_axis_name="s"),
    compiler_params=pltpu.CompilerParams(needs_layout_passes=False),
    scratch_types=[
        pltpu.VMEM((48 * EMB,), jnp.float32),
        pltpu.VMEM((SUPER + 16,), jnp.int32),
        pltpu.VMEM((SUPER + 16,), jnp.int32),
        pltpu.VMEM((SUPER + 16,), jnp.int32),
        pltpu.VMEM((SUPER + 16,), jnp.int32),
        pltpu.VMEM((SUPER + 16,), jnp.int32),
        pltpu.VMEM((SUPER + 16,), jnp.int32),
        pltpu.VMEM((SUPER + 16,), jnp.int32),
        pltpu.VMEM((SUPER + 16,), jnp.int32),
        pltpu.VMEM((SUPER + 16,), jnp.int32),
        pltpu.VMEM((SUPER * EMB,), jnp.float32),
        pltpu.VMEM((SUPER * EMB,), jnp.float32),
        pltpu.VMEM((SUPER * EMB,), jnp.float32),
        pltpu.SemaphoreType.DMA,
        pltpu.SemaphoreType.DMA,
        pltpu.SemaphoreType.DMA,
        pltpu.SemaphoreType.DMA,
        pltpu.SemaphoreType.DMA,
        pltpu.SemaphoreType.DMA,
        pltpu.SemaphoreType.DMA,
    ],
)(_sc_body)


@jax.jit
def kernel(edge_attr, W0, W1, W2):
    a = edge_attr.astype(jnp.int32)
    table = _build_table(W0, W1, W2).reshape(-1)
    flat = _sc_gather(a[:, 0], a[:, 1], a[:, 2], table)
    return flat.reshape(E, EMB)
